# baseline (speedup 1.0000x reference)
import functools

import jax
import jax.numpy as jnp
from jax import lax
from jax.experimental import pallas as pl
from jax.experimental.pallas import tpu as pltpu
from jax.experimental.pallas import tpu_sc as plsc

EMBED = 64
HIDDEN = 256
CW = 128
NC = 2
NS = 16


def _sc_gather(u2d, i2d, user_emb, item_emb):
    R = u2d.shape[0]
    nw = NC * NS
    rows_per_tile = R // nw
    B = R * CW
    mesh = plsc.VectorSubcoreMesh(core_axis_name="c", subcore_axis_name="s")

    @functools.partial(
        pl.kernel,
        out_type=(
            jax.ShapeDtypeStruct((B, EMBED), jnp.float32),
            jax.ShapeDtypeStruct((B, EMBED), jnp.float32),
        ),
        mesh=mesh,
        scratch_types=[
            pltpu.VMEM((rows_per_tile, CW), jnp.int32),
            pltpu.VMEM((rows_per_tile, CW), jnp.int32),
            pltpu.VMEM((rows_per_tile, CW, EMBED), jnp.float32),
            pltpu.VMEM((rows_per_tile, CW, EMBED), jnp.float32),
            pltpu.SemaphoreType.DMA,
        ],
        compiler_params=pltpu.CompilerParams(use_tc_tiling_on_sc=False),
    )
    def gather(u_hbm, i_hbm, ue_tab, ie_tab, ue_out, ie_out,
               uidx, iidx, urows, irows, sem):
        wid = lax.axis_index("s") * NC + lax.axis_index("c")
        r0 = wid * rows_per_tile
        pltpu.sync_copy(u_hbm.at[pl.ds(r0, rows_per_tile)], uidx)
        pltpu.sync_copy(i_hbm.at[pl.ds(r0, rows_per_tile)], iidx)
        cps = []
        for j in range(rows_per_tile):
            cps.append(pltpu.async_copy(ue_tab.at[uidx.at[j]], urows.at[j], sem))
            cps.append(pltpu.async_copy(ie_tab.at[iidx.at[j]], irows.at[j], sem))
        for c in cps:
            c.wait()
        for j in range(rows_per_tile):
            pltpu.sync_copy(urows.at[j], ue_out.at[pl.ds((r0 + j) * CW, CW)])
            pltpu.sync_copy(irows.at[j], ie_out.at[pl.ds((r0 + j) * CW, CW)])

    return gather(u2d, i2d, user_emb, item_emb)


def _mlp_body(ue_ref, ie_ref, w1t_ref, b1_ref, w2_ref, b2_ref, out_ref):
    w1t = w1t_ref[...]
    h = ue_ref[...] @ w1t[:EMBED] + ie_ref[...] @ w1t[EMBED:] + b1_ref[...]
    h = jnp.maximum(h, 0.0)
    out_ref[...] = jnp.sum(h * w2_ref[...], axis=1) + b2_ref[0]


def kernel(u, i, user_emb, item_emb, W1, b1, W2, b2):
    B = u.shape[0]
    R = B // CW
    u2d = u.reshape(R, CW).astype(jnp.int32)
    i2d = i.reshape(R, CW).astype(jnp.int32)

    ue, ie = _sc_gather(u2d, i2d, user_emb, item_emb)

    W1T = W1.T
    b1r = b1.reshape(1, HIDDEN)

    BLK = 2048
    nblk = B // BLK
    out = pl.pallas_call(
        _mlp_body,
        grid=(nblk,),
        in_specs=[
            pl.BlockSpec((BLK, EMBED), lambda b: (b, 0)),
            pl.BlockSpec((BLK, EMBED), lambda b: (b, 0)),
            pl.BlockSpec((2 * EMBED, HIDDEN), lambda b: (0, 0)),
            pl.BlockSpec((1, HIDDEN), lambda b: (0, 0)),
            pl.BlockSpec((1, HIDDEN), lambda b: (0, 0)),
            pl.BlockSpec(memory_space=pltpu.SMEM),
        ],
        out_specs=pl.BlockSpec((BLK,), lambda b: (b,)),
        out_shape=jax.ShapeDtypeStruct((B,), jnp.float32),
        compiler_params=pltpu.CompilerParams(
            dimension_semantics=("arbitrary",)),
    )(ue, ie, W1T, b1r, W2, b2)
    return out
